# baseline (speedup 1.0000x reference)
import functools

import jax
import jax.numpy as jnp
from jax import lax
from jax.experimental import pallas as pl
from jax.experimental.pallas import tpu as pltpu
from jax.experimental.pallas import tpu_sc as plsc

_H = 128
_NB = 6
_NW = 32
_AF = 136
_BIN = 148
_E = 320000
_N = 10000
_NM = 500
_APM = 20


def _make_gather_sum(R, C):
  rpw = R // _NW
  chunks = rpw // C
  assert rpw * _NW == R and chunks * C == rpw and C <= 128 and C % 8 == 0
  assert chunks >= 3 and chunks % 2 == 1

  mesh = plsc.VectorSubcoreMesh(core_axis_name="c", subcore_axis_name="s")

  @functools.partial(
      pl.kernel,
      out_type=jax.ShapeDtypeStruct((R, _H), jnp.float32),
      mesh=mesh,
      scratch_types=[
          pltpu.VMEM((_NB * rpw,), jnp.int32),
          [pltpu.VMEM((C, _H), jnp.float32) for _ in range(2)],
          [pltpu.SemaphoreType.DMA for _ in range(2)],
          [pltpu.SemaphoreType.DMA for _ in range(2)],
          [pltpu.SemaphoreType.DMA for _ in range(2)],
      ],
  )
  def gather_sum(table_hbm, idxF_hbm, out_hbm, idx_v, acc, semg, sema, sems):
    wid = lax.axis_index("s") * 2 + lax.axis_index("c")
    wbase = wid * rpw

    for k in range(_NB):
      pltpu.sync_copy(idxF_hbm.at[pl.ds(k * R + wbase, rpw)],
                      idx_v.at[pl.ds(k * rpw, rpw)])

    def fire_g0(c, b):
      return pltpu.async_copy(
          table_hbm.at[idx_v.at[pl.ds(c * C, C)]], acc[b], semg[b])

    def fire_adds(c, b):
      return [
          pltpu.async_copy(
              table_hbm.at[idx_v.at[pl.ds(k * rpw + c * C, C)]], acc[b],
              sema[b], add=True)
          for k in range(1, _NB)
      ]

    def fire_store(c, b):
      return pltpu.async_copy(
          acc[b], out_hbm.at[pl.ds(wbase + c * C, C)], sems[b])

    def wait_g0(b):
      pltpu.make_async_copy(table_hbm.at[idx_v.at[pl.ds(0, C)]], acc[b],
                            semg[b]).wait()

    def wait_adds(b):
      for _ in range(_NB - 1):
        pltpu.make_async_copy(table_hbm.at[idx_v.at[pl.ds(0, C)]], acc[b],
                              sema[b]).wait()

    def wait_store(b):
      pltpu.make_async_copy(acc[b], out_hbm.at[pl.ds(wbase, C)],
                            sems[b]).wait()

    fire_g0(0, 0)
    wait_g0(0)
    fire_adds(0, 0)
    fire_g0(1, 1)
    wait_adds(0)
    fire_store(0, 0)
    wait_g0(1)
    fire_adds(1, 1)
    wait_store(0)
    fire_g0(2, 0)
    wait_adds(1)
    fire_store(1, 1)

    def pair(t, carry):
      c0 = 2 * t
      for b in range(2):
        c = c0 + b
        wait_g0(b)
        fire_adds(c, b)
        wait_store(1 - b)
        fire_g0(c + 1, 1 - b)
        wait_adds(b)
        fire_store(c, b)
      return carry

    lax.fori_loop(1, chunks // 2, pair, 0)

    wait_g0(0)
    fire_adds(chunks - 1, 0)
    wait_adds(0)
    fire_store(chunks - 1, 0)
    wait_store(1)
    wait_store(0)

  return gather_sum


_E2 = _E // 2
_gather_sum_half = _make_gather_sum(_E2, 40)
_gather_sum_atoms = _make_gather_sum(10240, 64)


_BM = 2000


def _in_proj_body(fb_ref, w_ref, bin_ref, msg_ref):
  b = jnp.dot(fb_ref[...], w_ref[...], preferred_element_type=jnp.float32)
  bin_ref[...] = b
  msg_ref[...] = jnp.maximum(b, 0.0)


def _in_proj(fbonds, w_iT):
  return pl.pallas_call(
      _in_proj_body,
      grid=(_E // _BM,),
      in_specs=[
          pl.BlockSpec((_BM, _BIN), lambda i: (i, 0)),
          pl.BlockSpec((_BIN, _H), lambda i: (0, 0)),
      ],
      out_specs=[
          pl.BlockSpec((_BM, _H), lambda i: (i, 0)),
          pl.BlockSpec((_BM, _H), lambda i: (i, 0)),
      ],
      out_shape=[
          jax.ShapeDtypeStruct((_E, _H), jnp.float32),
          jax.ShapeDtypeStruct((_E, _H), jnp.float32),
      ],
  )(fbonds, w_iT)


def _update_a_body(bin_ref, s_ref, w_ref, out_ref):
  out_ref[...] = jnp.maximum(
      bin_ref[...]
      + jnp.dot(s_ref[...], w_ref[...], preferred_element_type=jnp.float32),
      0.0)


def _update_b_body(m_ref, bin_ref, s_ref, w_ref, out_ref):
  del m_ref
  out_ref[...] = jnp.maximum(
      bin_ref[...]
      + jnp.dot(s_ref[...], w_ref[...], preferred_element_type=jnp.float32),
      0.0)


_NBH = _E2 // _BM


def _update_a(binput, s_a, w_hT):
  return pl.pallas_call(
      _update_a_body,
      grid=(_NBH,),
      in_specs=[
          pl.BlockSpec((_BM, _H), lambda i: (i, 0)),
          pl.BlockSpec((_BM, _H), lambda i: (i, 0)),
          pl.BlockSpec((_H, _H), lambda i: (0, 0)),
      ],
      out_specs=pl.BlockSpec((_BM, _H), lambda i: (i, 0)),
      out_shape=jax.ShapeDtypeStruct((_E, _H), jnp.float32),
  )(binput, s_a, w_hT)


def _update_b(msg_buf, binput, s_b, w_hT):
  return pl.pallas_call(
      _update_b_body,
      grid=(_NBH,),
      in_specs=[
          pl.BlockSpec(memory_space=pl.ANY),
          pl.BlockSpec((_BM, _H), lambda i: (i + _NBH, 0)),
          pl.BlockSpec((_BM, _H), lambda i: (i, 0)),
          pl.BlockSpec((_H, _H), lambda i: (0, 0)),
      ],
      out_specs=pl.BlockSpec((_BM, _H), lambda i: (i + _NBH, 0)),
      out_shape=jax.ShapeDtypeStruct((_E, _H), jnp.float32),
      input_output_aliases={0: 0},
  )(msg_buf, binput, s_b, w_hT)


_BA = 2000
_MB = _BA // _APM


def _readout_body(fa_ref, s_ref, woa_ref, woh_ref, bo_ref, inv_ref, out_ref):
  h = (jnp.dot(fa_ref[...], woa_ref[...], preferred_element_type=jnp.float32)
       + jnp.dot(s_ref[...], woh_ref[...], preferred_element_type=jnp.float32)
       + bo_ref[...])
  h = jnp.maximum(h, 0.0)
  i = lax.broadcasted_iota(jnp.int32, (_MB, _BA), 0)
  j = lax.broadcasted_iota(jnp.int32, (_MB, _BA), 1)
  seg = jnp.where(j // _APM == i, 1.0, 0.0).astype(jnp.float32)
  mol = jnp.dot(seg, h, preferred_element_type=jnp.float32)
  out_ref[0] = mol * inv_ref[0]


def _readout(fatoms, s_a, w_oaT, w_ohT, b_o2, inv_sizes):
  out = pl.pallas_call(
      _readout_body,
      grid=(_N // _BA,),
      in_specs=[
          pl.BlockSpec((_BA, _AF), lambda i: (i, 0)),
          pl.BlockSpec((_BA, _H), lambda i: (i, 0)),
          pl.BlockSpec((_AF, _H), lambda i: (0, 0)),
          pl.BlockSpec((_H, _H), lambda i: (0, 0)),
          pl.BlockSpec((1, _H), lambda i: (0, 0)),
          pl.BlockSpec((1, _MB, 1), lambda i: (i, 0, 0)),
      ],
      out_specs=pl.BlockSpec((1, _MB, _H), lambda i: (i, 0, 0)),
      out_shape=jax.ShapeDtypeStruct((_NM // _MB, _MB, _H), jnp.float32),
  )(fatoms, s_a, w_oaT, w_ohT, b_o2, inv_sizes)
  return out.reshape(_NM, _H)


def kernel(fatoms, fbonds, agraph, bgraph, segment_ids, mol_sizes,
           W_i, W_h, W_o, b_o):
  del segment_ids
  w_iT = W_i.T
  w_hT = W_h.T
  w_oaT = W_o[:, :_AF].T
  w_ohT = W_o[:, _AF:].T
  b_o2 = b_o.reshape(1, _H)
  inv_sizes = (1.0 / mol_sizes).reshape(_NM // _MB, _MB, 1)
  bgraphTA = bgraph[:_E2].T.reshape(-1)
  bgraphTB = bgraph[_E2:].T.reshape(-1)
  agraphT = jnp.pad(agraph.T, ((0, 0), (0, 10240 - _N))).reshape(-1)

  binput, message = _in_proj(fbonds, w_iT)
  for _ in range(2):
    s_a = _gather_sum_half(message, bgraphTA)
    s_b = _gather_sum_half(message, bgraphTB)
    msg_buf = _update_a(binput, s_a, w_hT)
    message = _update_b(msg_buf, binput, s_b, w_hT)
  s_at = _gather_sum_atoms(message, agraphT)[:_N]
  return _readout(fatoms, s_at, w_oaT, w_ohT, b_o2, inv_sizes)


# speedup vs baseline: 1.0728x; 1.0728x over previous
import functools

import jax
import jax.numpy as jnp
from jax import lax
from jax.experimental import pallas as pl
from jax.experimental.pallas import tpu as pltpu
from jax.experimental.pallas import tpu_sc as plsc

_H = 128
_NB = 6
_NW = 32
_AF = 136
_BIN = 148
_E = 320000
_N = 10000
_NM = 500
_APM = 20


def _make_gather_sum(R, C):
  rpw = R // _NW
  full = rpw // C
  rem = rpw - full * C
  chunks = full + (1 if rem else 0)
  tail_n = rem if rem else C
  assert rpw * _NW == R and C <= 128 and C % 8 == 0 and tail_n % 8 == 0
  assert chunks >= 3 and chunks % 2 == 1

  mesh = plsc.VectorSubcoreMesh(core_axis_name="c", subcore_axis_name="s")

  @functools.partial(
      pl.kernel,
      out_type=jax.ShapeDtypeStruct((R, _H), jnp.float32),
      mesh=mesh,
      scratch_types=[
          pltpu.VMEM((_NB * rpw,), jnp.int32),
          [pltpu.VMEM((C, _H), jnp.float32) for _ in range(2)],
          [pltpu.SemaphoreType.DMA for _ in range(2)],
          [pltpu.SemaphoreType.DMA for _ in range(2)],
          [pltpu.SemaphoreType.DMA for _ in range(2)],
      ],
  )
  def gather_sum(table_hbm, idxF_hbm, out_hbm, idx_v, acc, semg, sema, sems):
    wid = lax.axis_index("s") * 2 + lax.axis_index("c")
    wbase = wid * rpw

    for k in range(_NB):
      pltpu.sync_copy(idxF_hbm.at[pl.ds(k * R + wbase, rpw)],
                      idx_v.at[pl.ds(k * rpw, rpw)])

    def fire_g0(c, b):
      return pltpu.async_copy(
          table_hbm.at[idx_v.at[pl.ds(c * C, C)]], acc[b], semg[b])

    def fire_adds(c, b, n=C):
      return [
          pltpu.async_copy(
              table_hbm.at[idx_v.at[pl.ds(k * rpw + c * C, n)]],
              acc[b].at[pl.ds(0, n)], sema[b], add=True)
          for k in range(1, _NB)
      ]

    def fire_store(c, b, n=C):
      return pltpu.async_copy(
          acc[b].at[pl.ds(0, n)], out_hbm.at[pl.ds(wbase + c * C, n)],
          sems[b])

    def wait_g0(b):
      pltpu.make_async_copy(table_hbm.at[idx_v.at[pl.ds(0, C)]], acc[b],
                            semg[b]).wait()

    def wait_adds(b, n=C):
      for _ in range(_NB - 1):
        pltpu.make_async_copy(table_hbm.at[idx_v.at[pl.ds(0, n)]],
                              acc[b].at[pl.ds(0, n)], sema[b]).wait()

    def wait_store(b, n=C):
      pltpu.make_async_copy(acc[b].at[pl.ds(0, n)],
                            out_hbm.at[pl.ds(wbase, n)], sems[b]).wait()

    fire_g0(0, 0)
    wait_g0(0)
    fire_adds(0, 0)
    fire_g0(1, 1)
    wait_adds(0)
    fire_store(0, 0)
    wait_g0(1)
    fire_adds(1, 1)
    wait_store(0)
    fire_g0(2, 0)
    wait_adds(1)
    fire_store(1, 1)

    def pair(t, carry):
      c0 = 2 * t
      for b in range(2):
        c = c0 + b
        wait_g0(b)
        fire_adds(c, b)
        wait_store(1 - b)
        fire_g0(c + 1, 1 - b)
        wait_adds(b)
        fire_store(c, b)
      return carry

    lax.fori_loop(1, chunks // 2, pair, 0)

    wait_g0(0)
    fire_adds(chunks - 1, 0, tail_n)
    wait_adds(0, tail_n)
    fire_store(chunks - 1, 0, tail_n)
    wait_store(1)
    wait_store(0, tail_n)

  return gather_sum


_E2 = _E // 2
_gather_sum_half = _make_gather_sum(_E2, 112)
_gather_sum_atoms = _make_gather_sum(10240, 64)


_BM = 2000


def _in_proj_body(fb_ref, w_ref, bin_ref, msg_ref):
  b = jnp.dot(fb_ref[...], w_ref[...], preferred_element_type=jnp.float32)
  bin_ref[...] = b
  msg_ref[...] = jnp.maximum(b, 0.0)


def _in_proj(fbonds, w_iT):
  return pl.pallas_call(
      _in_proj_body,
      grid=(_E // _BM,),
      in_specs=[
          pl.BlockSpec((_BM, _BIN), lambda i: (i, 0)),
          pl.BlockSpec((_BIN, _H), lambda i: (0, 0)),
      ],
      out_specs=[
          pl.BlockSpec((_BM, _H), lambda i: (i, 0)),
          pl.BlockSpec((_BM, _H), lambda i: (i, 0)),
      ],
      out_shape=[
          jax.ShapeDtypeStruct((_E, _H), jnp.float32),
          jax.ShapeDtypeStruct((_E, _H), jnp.float32),
      ],
  )(fbonds, w_iT)


def _update_a_body(bin_ref, s_ref, w_ref, out_ref):
  out_ref[...] = jnp.maximum(
      bin_ref[...]
      + jnp.dot(s_ref[...], w_ref[...], preferred_element_type=jnp.float32),
      0.0)


def _update_b_body(m_ref, bin_ref, s_ref, w_ref, out_ref):
  del m_ref
  out_ref[...] = jnp.maximum(
      bin_ref[...]
      + jnp.dot(s_ref[...], w_ref[...], preferred_element_type=jnp.float32),
      0.0)


_NBH = _E2 // _BM


def _update_a(binput, s_a, w_hT):
  return pl.pallas_call(
      _update_a_body,
      grid=(_NBH,),
      in_specs=[
          pl.BlockSpec((_BM, _H), lambda i: (i, 0)),
          pl.BlockSpec((_BM, _H), lambda i: (i, 0)),
          pl.BlockSpec((_H, _H), lambda i: (0, 0)),
      ],
      out_specs=pl.BlockSpec((_BM, _H), lambda i: (i, 0)),
      out_shape=jax.ShapeDtypeStruct((_E, _H), jnp.float32),
  )(binput, s_a, w_hT)


def _update_b(msg_buf, binput, s_b, w_hT):
  return pl.pallas_call(
      _update_b_body,
      grid=(_NBH,),
      in_specs=[
          pl.BlockSpec(memory_space=pl.ANY),
          pl.BlockSpec((_BM, _H), lambda i: (i + _NBH, 0)),
          pl.BlockSpec((_BM, _H), lambda i: (i, 0)),
          pl.BlockSpec((_H, _H), lambda i: (0, 0)),
      ],
      out_specs=pl.BlockSpec((_BM, _H), lambda i: (i + _NBH, 0)),
      out_shape=jax.ShapeDtypeStruct((_E, _H), jnp.float32),
      input_output_aliases={0: 0},
  )(msg_buf, binput, s_b, w_hT)


_BA = 2000
_MB = _BA // _APM


def _readout_body(fa_ref, s_ref, woa_ref, woh_ref, bo_ref, inv_ref, out_ref):
  h = (jnp.dot(fa_ref[...], woa_ref[...], preferred_element_type=jnp.float32)
       + jnp.dot(s_ref[...], woh_ref[...], preferred_element_type=jnp.float32)
       + bo_ref[...])
  h = jnp.maximum(h, 0.0)
  i = lax.broadcasted_iota(jnp.int32, (_MB, _BA), 0)
  j = lax.broadcasted_iota(jnp.int32, (_MB, _BA), 1)
  seg = jnp.where(j // _APM == i, 1.0, 0.0).astype(jnp.float32)
  mol = jnp.dot(seg, h, preferred_element_type=jnp.float32)
  out_ref[0] = mol * inv_ref[0]


def _readout(fatoms, s_a, w_oaT, w_ohT, b_o2, inv_sizes):
  out = pl.pallas_call(
      _readout_body,
      grid=(_N // _BA,),
      in_specs=[
          pl.BlockSpec((_BA, _AF), lambda i: (i, 0)),
          pl.BlockSpec((_BA, _H), lambda i: (i, 0)),
          pl.BlockSpec((_AF, _H), lambda i: (0, 0)),
          pl.BlockSpec((_H, _H), lambda i: (0, 0)),
          pl.BlockSpec((1, _H), lambda i: (0, 0)),
          pl.BlockSpec((1, _MB, 1), lambda i: (i, 0, 0)),
      ],
      out_specs=pl.BlockSpec((1, _MB, _H), lambda i: (i, 0, 0)),
      out_shape=jax.ShapeDtypeStruct((_NM // _MB, _MB, _H), jnp.float32),
  )(fatoms, s_a, w_oaT, w_ohT, b_o2, inv_sizes)
  return out.reshape(_NM, _H)


def kernel(fatoms, fbonds, agraph, bgraph, segment_ids, mol_sizes,
           W_i, W_h, W_o, b_o):
  del segment_ids
  w_iT = W_i.T
  w_hT = W_h.T
  w_oaT = W_o[:, :_AF].T
  w_ohT = W_o[:, _AF:].T
  b_o2 = b_o.reshape(1, _H)
  inv_sizes = (1.0 / mol_sizes).reshape(_NM // _MB, _MB, 1)
  bgraphTA = bgraph[:_E2].T.reshape(-1)
  bgraphTB = bgraph[_E2:].T.reshape(-1)
  agraphT = jnp.pad(agraph.T, ((0, 0), (0, 10240 - _N))).reshape(-1)

  binput, message = _in_proj(fbonds, w_iT)
  for _ in range(2):
    s_a = _gather_sum_half(message, bgraphTA)
    s_b = _gather_sum_half(message, bgraphTB)
    msg_buf = _update_a(binput, s_a, w_hT)
    message = _update_b(msg_buf, binput, s_b, w_hT)
  s_at = _gather_sum_atoms(message, agraphT)[:_N]
  return _readout(fatoms, s_at, w_oaT, w_ohT, b_o2, inv_sizes)


# speedup vs baseline: 1.1148x; 1.0392x over previous
import functools

import jax
import jax.numpy as jnp
from jax import lax
from jax.experimental import pallas as pl
from jax.experimental.pallas import tpu as pltpu
from jax.experimental.pallas import tpu_sc as plsc

_H = 128
_NB = 6
_NW = 32
_AF = 136
_BIN = 148
_E = 320000
_N = 10000
_NM = 500
_APM = 20


def _make_gather_sum(R, C):
  rpw = R // _NW
  full = rpw // C
  rem = rpw - full * C
  chunks = full + (1 if rem else 0)
  tail_n = rem if rem else C
  assert rpw * _NW == R and C <= 128 and C % 8 == 0 and tail_n % 8 == 0
  assert chunks >= 3 and chunks % 2 == 1

  mesh = plsc.VectorSubcoreMesh(core_axis_name="c", subcore_axis_name="s")

  @functools.partial(
      pl.kernel,
      out_type=jax.ShapeDtypeStruct((R, _H), jnp.float32),
      mesh=mesh,
      scratch_types=[
          pltpu.VMEM((_NB * rpw,), jnp.int32),
          [pltpu.VMEM((C, _H), jnp.float32) for _ in range(2)],
          [pltpu.SemaphoreType.DMA for _ in range(2)],
          [pltpu.SemaphoreType.DMA for _ in range(2)],
          [pltpu.SemaphoreType.DMA for _ in range(2)],
      ],
  )
  def gather_sum(table_hbm, idxF_hbm, out_hbm, idx_v, acc, semg, sema, sems):
    wid = lax.axis_index("s") * 2 + lax.axis_index("c")
    wbase = wid * rpw

    for k in range(_NB):
      pltpu.sync_copy(idxF_hbm.at[pl.ds(k * R + wbase, rpw)],
                      idx_v.at[pl.ds(k * rpw, rpw)])

    def fire_g0(c, b):
      return pltpu.async_copy(
          table_hbm.at[idx_v.at[pl.ds(c * C, C)]], acc[b], semg[b])

    def fire_adds(c, b, n=C):
      return [
          pltpu.async_copy(
              table_hbm.at[idx_v.at[pl.ds(k * rpw + c * C, n)]],
              acc[b].at[pl.ds(0, n)], sema[b], add=True)
          for k in range(1, _NB)
      ]

    def fire_store(c, b, n=C):
      return pltpu.async_copy(
          acc[b].at[pl.ds(0, n)], out_hbm.at[pl.ds(wbase + c * C, n)],
          sems[b])

    def wait_g0(b):
      pltpu.make_async_copy(table_hbm.at[idx_v.at[pl.ds(0, C)]], acc[b],
                            semg[b]).wait()

    def wait_adds(b, n=C):
      for _ in range(_NB - 1):
        pltpu.make_async_copy(table_hbm.at[idx_v.at[pl.ds(0, n)]],
                              acc[b].at[pl.ds(0, n)], sema[b]).wait()

    def wait_store(b, n=C):
      pltpu.make_async_copy(acc[b].at[pl.ds(0, n)],
                            out_hbm.at[pl.ds(wbase, n)], sems[b]).wait()

    fire_g0(0, 0)
    wait_g0(0)
    fire_adds(0, 0)
    fire_g0(1, 1)
    wait_adds(0)
    fire_store(0, 0)
    wait_g0(1)
    fire_adds(1, 1)
    wait_store(0)
    fire_g0(2, 0)
    wait_adds(1)
    fire_store(1, 1)

    def pair(t, carry):
      c0 = 2 * t
      for b in range(2):
        c = c0 + b
        wait_g0(b)
        fire_adds(c, b)
        wait_store(1 - b)
        fire_g0(c + 1, 1 - b)
        wait_adds(b)
        fire_store(c, b)
      return carry

    lax.fori_loop(1, chunks // 2, pair, 0)

    wait_g0(0)
    fire_adds(chunks - 1, 0, tail_n)
    wait_adds(0, tail_n)
    fire_store(chunks - 1, 0, tail_n)
    wait_store(1)
    wait_store(0, tail_n)

  return gather_sum


_E2 = _E // 2
_gather_sum_half = _make_gather_sum(_E2, 112)
_gather_sum_atoms = _make_gather_sum(10240, 64)


_BM = 4000


def _in_proj_body(fb_ref, w_ref, bin_ref, msg_ref):
  b = jnp.dot(fb_ref[...], w_ref[...], preferred_element_type=jnp.float32)
  bin_ref[...] = b
  msg_ref[...] = jnp.maximum(b, 0.0)


def _in_proj(fbonds, w_iT):
  return pl.pallas_call(
      _in_proj_body,
      grid=(_E // _BM,),
      in_specs=[
          pl.BlockSpec((_BM, _BIN), lambda i: (i, 0)),
          pl.BlockSpec((_BIN, _H), lambda i: (0, 0)),
      ],
      out_specs=[
          pl.BlockSpec((_BM, _H), lambda i: (i, 0)),
          pl.BlockSpec((_BM, _H), lambda i: (i, 0)),
      ],
      out_shape=[
          jax.ShapeDtypeStruct((_E, _H), jnp.float32),
          jax.ShapeDtypeStruct((_E, _H), jnp.float32),
      ],
  )(fbonds, w_iT)


def _update_a_body(bin_ref, s_ref, w_ref, out_ref):
  out_ref[...] = jnp.maximum(
      bin_ref[...]
      + jnp.dot(s_ref[...], w_ref[...], preferred_element_type=jnp.float32),
      0.0)


def _update_b_body(m_ref, bin_ref, s_ref, w_ref, out_ref):
  del m_ref
  out_ref[...] = jnp.maximum(
      bin_ref[...]
      + jnp.dot(s_ref[...], w_ref[...], preferred_element_type=jnp.float32),
      0.0)


_NBH = _E2 // _BM


def _update_a(binput, s_a, w_hT):
  return pl.pallas_call(
      _update_a_body,
      grid=(_NBH,),
      in_specs=[
          pl.BlockSpec((_BM, _H), lambda i: (i, 0)),
          pl.BlockSpec((_BM, _H), lambda i: (i, 0)),
          pl.BlockSpec((_H, _H), lambda i: (0, 0)),
      ],
      out_specs=pl.BlockSpec((_BM, _H), lambda i: (i, 0)),
      out_shape=jax.ShapeDtypeStruct((_E, _H), jnp.float32),
  )(binput, s_a, w_hT)


def _update_b(msg_buf, binput, s_b, w_hT):
  return pl.pallas_call(
      _update_b_body,
      grid=(_NBH,),
      in_specs=[
          pl.BlockSpec(memory_space=pl.ANY),
          pl.BlockSpec((_BM, _H), lambda i: (i + _NBH, 0)),
          pl.BlockSpec((_BM, _H), lambda i: (i, 0)),
          pl.BlockSpec((_H, _H), lambda i: (0, 0)),
      ],
      out_specs=pl.BlockSpec((_BM, _H), lambda i: (i + _NBH, 0)),
      out_shape=jax.ShapeDtypeStruct((_E, _H), jnp.float32),
      input_output_aliases={0: 0},
  )(msg_buf, binput, s_b, w_hT)


_BA = 2000
_MB = _BA // _APM


def _readout_body(fa_ref, s_ref, woa_ref, woh_ref, bo_ref, inv_ref, out_ref):
  h = (jnp.dot(fa_ref[...], woa_ref[...], preferred_element_type=jnp.float32)
       + jnp.dot(s_ref[...], woh_ref[...], preferred_element_type=jnp.float32)
       + bo_ref[...])
  h = jnp.maximum(h, 0.0)
  i = lax.broadcasted_iota(jnp.int32, (_MB, _BA), 0)
  j = lax.broadcasted_iota(jnp.int32, (_MB, _BA), 1)
  seg = jnp.where(j // _APM == i, 1.0, 0.0).astype(jnp.float32)
  mol = jnp.dot(seg, h, preferred_element_type=jnp.float32)
  out_ref[0] = mol * inv_ref[0]


def _readout(fatoms, s_a, w_oaT, w_ohT, b_o2, inv_sizes):
  out = pl.pallas_call(
      _readout_body,
      grid=(_N // _BA,),
      in_specs=[
          pl.BlockSpec((_BA, _AF), lambda i: (i, 0)),
          pl.BlockSpec((_BA, _H), lambda i: (i, 0)),
          pl.BlockSpec((_AF, _H), lambda i: (0, 0)),
          pl.BlockSpec((_H, _H), lambda i: (0, 0)),
          pl.BlockSpec((1, _H), lambda i: (0, 0)),
          pl.BlockSpec((1, _MB, 1), lambda i: (i, 0, 0)),
      ],
      out_specs=pl.BlockSpec((1, _MB, _H), lambda i: (i, 0, 0)),
      out_shape=jax.ShapeDtypeStruct((_NM // _MB, _MB, _H), jnp.float32),
  )(fatoms, s_a, w_oaT, w_ohT, b_o2, inv_sizes)
  return out.reshape(_NM, _H)


def kernel(fatoms, fbonds, agraph, bgraph, segment_ids, mol_sizes,
           W_i, W_h, W_o, b_o):
  del segment_ids
  w_iT = W_i.T
  w_hT = W_h.T
  w_oaT = W_o[:, :_AF].T
  w_ohT = W_o[:, _AF:].T
  b_o2 = b_o.reshape(1, _H)
  inv_sizes = (1.0 / mol_sizes).reshape(_NM // _MB, _MB, 1)
  bgraphTA = bgraph[:_E2].T.reshape(-1)
  bgraphTB = bgraph[_E2:].T.reshape(-1)
  agraphT = jnp.pad(agraph.T, ((0, 0), (0, 10240 - _N))).reshape(-1)

  binput, message = _in_proj(fbonds, w_iT)
  for _ in range(2):
    s_a = _gather_sum_half(message, bgraphTA)
    s_b = _gather_sum_half(message, bgraphTB)
    msg_buf = _update_a(binput, s_a, w_hT)
    message = _update_b(msg_buf, binput, s_b, w_hT)
  s_at = _gather_sum_atoms(message, agraphT)[:_N]
  return _readout(fatoms, s_at, w_oaT, w_ohT, b_o2, inv_sizes)
